# baseline (speedup 1.0000x reference)
import functools

import jax
import jax.numpy as jnp
from jax import lax
from jax.experimental import pallas as pl
from jax.experimental.pallas import tpu as pltpu
from jax.experimental.pallas import tpu_sc as plsc

N_NODES = 10000
D = 512
B = 2048
K = 32
L = 16
NW = 32
RPW = B // NW
R = 4
NCH = RPW // R
NPAIR = NCH // 2
G = D // (2 * L)


def _sc_body(feat_hbm, ids_hbm, self_hbm, out_hbm,
             ids_v, table_v, nd_v, inv_v,
             rows0, rows1, self0, self1, out0, out1,
             sem_r0, sem_r1, sem_s0, sem_s1, sem_o0, sem_o1):
    cid = lax.axis_index("c")
    sid = lax.axis_index("s")
    wid = sid * 2 + cid
    row0 = wid * RPW

    pltpu.sync_copy(ids_hbm.at[pl.ds(row0 * K, RPW * K)], ids_v)

    pos_a = lax.iota(jnp.int32, L)
    pos_b = pos_a + L
    zf = jnp.zeros((L,), jnp.float32)
    zi = jnp.zeros((L,), jnp.int32)

    def dd_body(r, carry):
        base = r * K
        a = ids_v[pl.ds(base, L)]
        b = ids_v[pl.ds(base + L, L)]
        plsc.store_scatter(table_v, [a], pos_a)
        plsc.store_scatter(table_v, [b], pos_b)
        ga = plsc.load_gather(table_v, [a])
        gb = plsc.load_gather(table_v, [b])
        fa = ga == pos_a
        fb = gb == pos_b
        id0 = plsc.load_gather(ids_v, [zi + base])
        ids_v[pl.ds(base, L)] = jnp.where(fa, a, id0)
        ids_v[pl.ds(base + L, L)] = jnp.where(fb, b, id0)
        fa_f = jnp.where(fa, 1.0, 0.0).astype(jnp.float32)
        fb_f = jnp.where(fb, 1.0, 0.0).astype(jnp.float32)
        n_unique = jnp.sum(fa_f) + jnp.sum(fb_f)
        nd_v[r, pl.ds(0, L)] = (K - n_unique) + zf
        inv_v[r, pl.ds(0, L)] = (1.0 + zf) / (n_unique + zf)
        return carry

    lax.fori_loop(0, RPW, dd_body, 0)

    def rows_dma(ch, buf, sem):
        idx = ids_v.at[pl.ds(ch * R * K, R * K)]
        return pltpu.make_async_copy(feat_hbm.at[idx], buf, sem)

    def self_dma(ch, buf, sem):
        return pltpu.make_async_copy(
            self_hbm.at[pl.ds(row0 + ch * R, R)], buf, sem)

    def out_dma(ch, buf, sem):
        return pltpu.make_async_copy(
            buf, out_hbm.at[pl.ds(row0 + ch * R, R)], sem)

    def load_groups(rows_b, row):
        out = []
        for g in range(G):
            w = rows_b[row, pl.ds(g * L, L)]
            ab = plsc.bitcast(w, jnp.bfloat16)
            a, b = plsc.unpack(ab, format=plsc.PackFormat.INTERLEAVED,
                               preferred_element_type=jnp.float32)
            out.append(a)
            out.append(b)
        return out

    def compute(ch, rows_b, self_b, out_b):
        for rr in range(R):
            def acc_body(j, acc):
                vals = load_groups(rows_b, rr * K + j)
                return tuple(acc[c] + vals[c] for c in range(2 * G))

            acc0 = tuple(jnp.zeros((L,), jnp.float32) for _ in range(2 * G))
            acc = lax.fori_loop(0, K, acc_body, acc0)

            r = ch * R + rr
            nd = nd_v[r, pl.ds(0, L)]
            inv = inv_v[r, pl.ds(0, L)]
            r0v = load_groups(rows_b, rr * K)
            rvec = zi + rr
            for g in range(G):
                agg_e = (acc[2 * g] - nd * r0v[2 * g]) * inv
                agg_o = (acc[2 * g + 1] - nd * r0v[2 * g + 1]) * inv
                col_e = 2 * pos_a + (32 * g)
                col_o = col_e + 1
                sf_e = plsc.load_gather(self_b, [rvec, col_e])
                sf_o = plsc.load_gather(self_b, [rvec, col_o])
                plsc.store_scatter(out_b, [rvec, col_e], sf_e - agg_e)
                plsc.store_scatter(out_b, [rvec, col_o], sf_o - agg_o)
                plsc.store_scatter(out_b, [rvec, col_e + D], agg_e)
                plsc.store_scatter(out_b, [rvec, col_o + D], agg_o)

    rows_dma(0, rows0, sem_r0).start()
    self_dma(0, self0, sem_s0).start()

    def pair_body(i2, carry):
        ch0 = i2 * 2
        ch1 = ch0 + 1
        rows_dma(ch1, rows1, sem_r1).start()
        self_dma(ch1, self1, sem_s1).start()

        rows_dma(ch0, rows0, sem_r0).wait()
        self_dma(ch0, self0, sem_s0).wait()

        @pl.when(i2 > 0)
        def _():
            out_dma(ch0 - 2, out0, sem_o0).wait()

        compute(ch0, rows0, self0, out0)
        out_dma(ch0, out0, sem_o0).start()

        @pl.when(i2 < NPAIR - 1)
        def _():
            rows_dma(ch0 + 2, rows0, sem_r0).start()
            self_dma(ch0 + 2, self0, sem_s0).start()

        rows_dma(ch1, rows1, sem_r1).wait()
        self_dma(ch1, self1, sem_s1).wait()

        @pl.when(i2 > 0)
        def _():
            out_dma(ch1 - 2, out1, sem_o1).wait()

        compute(ch1, rows1, self1, out1)
        out_dma(ch1, out1, sem_o1).start()
        return carry

    lax.fori_loop(0, NPAIR, pair_body, 0)
    out_dma(NCH - 2, out0, sem_o0).wait()
    out_dma(NCH - 1, out1, sem_o1).wait()


@jax.jit
def _intra_agg(feat_pre, ids_flat, self_feats):
    mesh = plsc.VectorSubcoreMesh(core_axis_name="c", subcore_axis_name="s")
    f = functools.partial(
        pl.kernel,
        mesh=mesh,
        compiler_params=pltpu.CompilerParams(needs_layout_passes=False),
        out_type=jax.ShapeDtypeStruct((B, 2 * D), jnp.float32),
        scratch_types=[
            pltpu.VMEM((RPW * K,), jnp.int32),
            pltpu.VMEM((N_NODES,), jnp.int32),
            pltpu.VMEM((RPW, L), jnp.float32),
            pltpu.VMEM((RPW, L), jnp.float32),
            pltpu.VMEM((R * K, D // 2), jnp.int32),
            pltpu.VMEM((R * K, D // 2), jnp.int32),
            pltpu.VMEM((R, D), jnp.float32),
            pltpu.VMEM((R, D), jnp.float32),
            pltpu.VMEM((R, 2 * D), jnp.float32),
            pltpu.VMEM((R, 2 * D), jnp.float32),
            pltpu.SemaphoreType.DMA,
            pltpu.SemaphoreType.DMA,
            pltpu.SemaphoreType.DMA,
            pltpu.SemaphoreType.DMA,
            pltpu.SemaphoreType.DMA,
            pltpu.SemaphoreType.DMA,
        ],
    )(_sc_body)
    return f(feat_pre, ids_flat, self_feats)


def kernel(features, nodes, to_neighs_list, self_feats):
    del nodes
    ids_flat = to_neighs_list.astype(jnp.int32).reshape(-1)
    fp = features.astype(jnp.bfloat16).reshape(N_NODES, D // 2, 2)
    fp = lax.bitcast_convert_type(fp, jnp.int32)
    return _intra_agg(fp, ids_flat, self_feats)


# speedup vs baseline: 1.0401x; 1.0401x over previous
import functools

import jax
import jax.numpy as jnp
from jax import lax
from jax.experimental import pallas as pl
from jax.experimental.pallas import tpu as pltpu
from jax.experimental.pallas import tpu_sc as plsc

N_NODES = 10000
D = 512
B = 2048
K = 32
L = 16
NW = 32
RPW = B // NW
R = 4
NCH = RPW // R
NPAIR = NCH // 2
G = D // (2 * L)


def _vperm(x, idx):
    dnums = lax.GatherDimensionNumbers(
        offset_dims=(), collapsed_slice_dims=(0,), start_index_map=(0,))
    return lax.gather(x, idx[:, None], dnums, slice_sizes=(1,),
                      mode=lax.GatherScatterMode.PROMISE_IN_BOUNDS)


def _sc_body(feat_hbm, ids_hbm, self_hbm, out_hbm,
             ids_v, table_v, nd_v, inv_v,
             rows0, rows1, self0, self1, out0, out1,
             sem_r0, sem_r1, sem_s0, sem_s1, sem_o0, sem_o1):
    cid = lax.axis_index("c")
    sid = lax.axis_index("s")
    wid = sid * 2 + cid
    row0 = wid * RPW

    pltpu.sync_copy(ids_hbm.at[pl.ds(row0 * K, RPW * K)], ids_v)

    pos_a = lax.iota(jnp.int32, L)
    pos_b = pos_a + L
    zf = jnp.zeros((L,), jnp.float32)
    zi = jnp.zeros((L,), jnp.int32)
    even_lane = (pos_a & 1) == 0
    idx_lo = pos_a >> 1
    idx_hi = (pos_a + L) >> 1

    def dd_body(r, carry):
        base = r * K
        a = ids_v[pl.ds(base, L)]
        b = ids_v[pl.ds(base + L, L)]
        plsc.store_scatter(table_v, [a], pos_a)
        plsc.store_scatter(table_v, [b], pos_b)
        ga = plsc.load_gather(table_v, [a])
        gb = plsc.load_gather(table_v, [b])
        fa = ga == pos_a
        fb = gb == pos_b
        id0 = plsc.load_gather(ids_v, [zi + base])
        ids_v[pl.ds(base, L)] = jnp.where(fa, a, id0)
        ids_v[pl.ds(base + L, L)] = jnp.where(fb, b, id0)
        fa_f = jnp.where(fa, 1.0, 0.0).astype(jnp.float32)
        fb_f = jnp.where(fb, 1.0, 0.0).astype(jnp.float32)
        n_unique = jnp.sum(fa_f) + jnp.sum(fb_f)
        nd_v[r, pl.ds(0, L)] = (K - n_unique) + zf
        inv_v[r, pl.ds(0, L)] = (1.0 + zf) / (n_unique + zf)
        return carry

    lax.fori_loop(0, RPW, dd_body, 0)

    def rows_dma(ch, buf, sem):
        idx = ids_v.at[pl.ds(ch * R * K, R * K)]
        return pltpu.make_async_copy(feat_hbm.at[idx], buf, sem)

    def self_dma(ch, buf, sem):
        return pltpu.make_async_copy(
            self_hbm.at[pl.ds(row0 + ch * R, R)], buf, sem)

    def out_dma(ch, buf, sem):
        return pltpu.make_async_copy(
            buf, out_hbm.at[pl.ds(row0 + ch * R, R)], sem)

    def load_groups(rows_b, row):
        out = []
        for g in range(G):
            w = rows_b[row, pl.ds(g * L, L)]
            ab = plsc.bitcast(w, jnp.bfloat16)
            a, b = plsc.unpack(ab, format=plsc.PackFormat.INTERLEAVED,
                               preferred_element_type=jnp.float32)
            out.append(a)
            out.append(b)
        return out

    def compute(ch, rows_b, self_b, out_b):
        for rr in range(R):
            def acc_body(j, acc):
                vals = load_groups(rows_b, rr * K + j)
                return tuple(acc[c] + vals[c] for c in range(2 * G))

            acc0 = tuple(jnp.zeros((L,), jnp.float32) for _ in range(2 * G))
            acc = lax.fori_loop(0, K, acc_body, acc0)

            r = ch * R + rr
            nd = nd_v[r, pl.ds(0, L)]
            inv = inv_v[r, pl.ds(0, L)]
            r0v = load_groups(rows_b, rr * K)
            for g in range(G):
                agg_e = (acc[2 * g] - nd * r0v[2 * g]) * inv
                agg_o = (acc[2 * g + 1] - nd * r0v[2 * g + 1]) * inv
                agg_lo = jnp.where(even_lane, _vperm(agg_e, idx_lo),
                                   _vperm(agg_o, idx_lo))
                agg_hi = jnp.where(even_lane, _vperm(agg_e, idx_hi),
                                   _vperm(agg_o, idx_hi))
                c0 = 32 * g
                sf_lo = self_b[rr, pl.ds(c0, L)]
                sf_hi = self_b[rr, pl.ds(c0 + L, L)]
                out_b[rr, pl.ds(c0, L)] = sf_lo - agg_lo
                out_b[rr, pl.ds(c0 + L, L)] = sf_hi - agg_hi
                out_b[rr, pl.ds(D + c0, L)] = agg_lo
                out_b[rr, pl.ds(D + c0 + L, L)] = agg_hi

    rows_dma(0, rows0, sem_r0).start()
    self_dma(0, self0, sem_s0).start()

    def pair_body(i2, carry):
        ch0 = i2 * 2
        ch1 = ch0 + 1
        rows_dma(ch1, rows1, sem_r1).start()
        self_dma(ch1, self1, sem_s1).start()

        rows_dma(ch0, rows0, sem_r0).wait()
        self_dma(ch0, self0, sem_s0).wait()

        @pl.when(i2 > 0)
        def _():
            out_dma(ch0 - 2, out0, sem_o0).wait()

        compute(ch0, rows0, self0, out0)
        out_dma(ch0, out0, sem_o0).start()

        @pl.when(i2 < NPAIR - 1)
        def _():
            rows_dma(ch0 + 2, rows0, sem_r0).start()
            self_dma(ch0 + 2, self0, sem_s0).start()

        rows_dma(ch1, rows1, sem_r1).wait()
        self_dma(ch1, self1, sem_s1).wait()

        @pl.when(i2 > 0)
        def _():
            out_dma(ch1 - 2, out1, sem_o1).wait()

        compute(ch1, rows1, self1, out1)
        out_dma(ch1, out1, sem_o1).start()
        return carry

    lax.fori_loop(0, NPAIR, pair_body, 0)
    out_dma(NCH - 2, out0, sem_o0).wait()
    out_dma(NCH - 1, out1, sem_o1).wait()


@jax.jit
def _intra_agg(feat_pre, ids_flat, self_feats):
    mesh = plsc.VectorSubcoreMesh(core_axis_name="c", subcore_axis_name="s")
    f = functools.partial(
        pl.kernel,
        mesh=mesh,
        compiler_params=pltpu.CompilerParams(needs_layout_passes=False),
        out_type=jax.ShapeDtypeStruct((B, 2 * D), jnp.float32),
        scratch_types=[
            pltpu.VMEM((RPW * K,), jnp.int32),
            pltpu.VMEM((N_NODES,), jnp.int32),
            pltpu.VMEM((RPW, L), jnp.float32),
            pltpu.VMEM((RPW, L), jnp.float32),
            pltpu.VMEM((R * K, D // 2), jnp.int32),
            pltpu.VMEM((R * K, D // 2), jnp.int32),
            pltpu.VMEM((R, D), jnp.float32),
            pltpu.VMEM((R, D), jnp.float32),
            pltpu.VMEM((R, 2 * D), jnp.float32),
            pltpu.VMEM((R, 2 * D), jnp.float32),
            pltpu.SemaphoreType.DMA,
            pltpu.SemaphoreType.DMA,
            pltpu.SemaphoreType.DMA,
            pltpu.SemaphoreType.DMA,
            pltpu.SemaphoreType.DMA,
            pltpu.SemaphoreType.DMA,
        ],
    )(_sc_body)
    return f(feat_pre, ids_flat, self_feats)


def kernel(features, nodes, to_neighs_list, self_feats):
    del nodes
    ids_flat = to_neighs_list.astype(jnp.int32).reshape(-1)
    fp = features.astype(jnp.bfloat16).reshape(N_NODES, D // 2, 2)
    fp = lax.bitcast_convert_type(fp, jnp.int32)
    return _intra_agg(fp, ids_flat, self_feats)


# speedup vs baseline: 3.3581x; 3.2287x over previous
import functools

import jax
import jax.numpy as jnp
from jax import lax
from jax.experimental import pallas as pl
from jax.experimental.pallas import tpu as pltpu
from jax.experimental.pallas import tpu_sc as plsc

N_NODES = 10000
D = 512
B = 2048
K = 32
L = 16
NW = 32
RPW = B // NW
R = 2
NCH = RPW // R
NPAIR = NCH // 2
CPD = D // L


def _sc_body(feat_hbm, ids_hbm, self_hbm, out_hbm,
             ids_v, table_v, nd_v, inv_v,
             rows0, rows1, self0, self1, out0, out1,
             sem_r0, sem_r0b, sem_r1, sem_r1b,
             sem_s0, sem_s1, sem_o0, sem_o1):
    cid = lax.axis_index("c")
    sid = lax.axis_index("s")
    wid = sid * 2 + cid
    row0 = wid * RPW

    pltpu.sync_copy(ids_hbm.at[pl.ds(row0 * K, RPW * K)], ids_v)

    pos_a = lax.iota(jnp.int32, L)
    pos_b = pos_a + L
    zf = jnp.zeros((L,), jnp.float32)
    zi = jnp.zeros((L,), jnp.int32)

    def dd_body(r, carry):
        base = r * K
        a = ids_v[pl.ds(base, L)]
        b = ids_v[pl.ds(base + L, L)]
        plsc.store_scatter(table_v, [a], pos_a)
        plsc.store_scatter(table_v, [b], pos_b)
        ga = plsc.load_gather(table_v, [a])
        gb = plsc.load_gather(table_v, [b])
        fa = ga == pos_a
        fb = gb == pos_b
        id0 = plsc.load_gather(ids_v, [zi + base])
        ids_v[pl.ds(base, L)] = jnp.where(fa, a, id0)
        ids_v[pl.ds(base + L, L)] = jnp.where(fb, b, id0)
        fa_f = jnp.where(fa, 1.0, 0.0).astype(jnp.float32)
        fb_f = jnp.where(fb, 1.0, 0.0).astype(jnp.float32)
        n_unique = jnp.sum(fa_f) + jnp.sum(fb_f)
        nd_v[r, pl.ds(0, L)] = (K - n_unique) + zf
        inv_v[r, pl.ds(0, L)] = (1.0 + zf) / (n_unique + zf)
        return carry

    lax.fori_loop(0, RPW, dd_body, 0)

    H = R * K // 2

    def rows_dma_h(ch, half, buf, sem):
        idx = ids_v.at[pl.ds(ch * R * K + half * H, H)]
        return pltpu.make_async_copy(
            feat_hbm.at[idx], buf.at[pl.ds(half * H, H)], sem)

    class _RowsPair:
        def __init__(self, ch, buf, sems):
            self.parts = [rows_dma_h(ch, 0, buf, sems[0]),
                          rows_dma_h(ch, 1, buf, sems[1])]

        def start(self):
            for p in self.parts:
                p.start()

        def wait(self):
            for p in self.parts:
                p.wait()

    def rows_dma(ch, buf, sem):
        return _RowsPair(ch, buf, sem)

    def self_dma(ch, buf, sem):
        return pltpu.make_async_copy(
            self_hbm.at[pl.ds(row0 + ch * R, R)], buf, sem)

    UNROLL = 1

    def compute(ch, rows_b, self_b, out_b):
        for rr in range(R):
            def acc_body(jj, acc):
                for u in range(UNROLL):
                    rbase = rr * K + jj * UNROLL + u
                    acc = tuple(acc[c] + rows_b[rbase, pl.ds(c * L, L)]
                                for c in range(CPD))
                return acc

            acc0 = tuple(jnp.zeros((L,), jnp.float32) for _ in range(CPD))
            acc = lax.fori_loop(0, K // UNROLL, acc_body, acc0)

            r = ch * R + rr
            nd = nd_v[r, pl.ds(0, L)]
            inv = inv_v[r, pl.ds(0, L)]
            for c in range(CPD):
                r0c = rows_b[rr * K, pl.ds(c * L, L)]
                aggc = (acc[c] - nd * r0c) * inv
                out_b[rr, pl.ds(c * L, L)] = self_b[rr, pl.ds(c * L, L)] - aggc
                out_b[rr, pl.ds(D + c * L, L)] = aggc

    def out_dma(ch, buf, sem):
        return pltpu.make_async_copy(
            buf, out_hbm.at[pl.ds(row0 + ch * R, R)], sem)

    rows_dma(0, rows0, (sem_r0, sem_r0b)).start()
    self_dma(0, self0, sem_s0).start()

    def pair_body(i2, carry):
        ch0 = i2 * 2
        ch1 = ch0 + 1
        rows_dma(ch1, rows1, (sem_r1, sem_r1b)).start()
        self_dma(ch1, self1, sem_s1).start()

        rows_dma(ch0, rows0, (sem_r0, sem_r0b)).wait()
        self_dma(ch0, self0, sem_s0).wait()

        @pl.when(i2 > 0)
        def _():
            out_dma(ch0 - 2, out0, sem_o0).wait()

        compute(ch0, rows0, self0, out0)
        out_dma(ch0, out0, sem_o0).start()

        @pl.when(i2 < NPAIR - 1)
        def _():
            rows_dma(ch0 + 2, rows0, (sem_r0, sem_r0b)).start()
            self_dma(ch0 + 2, self0, sem_s0).start()

        rows_dma(ch1, rows1, (sem_r1, sem_r1b)).wait()
        self_dma(ch1, self1, sem_s1).wait()

        @pl.when(i2 > 0)
        def _():
            out_dma(ch1 - 2, out1, sem_o1).wait()

        compute(ch1, rows1, self1, out1)
        out_dma(ch1, out1, sem_o1).start()
        return carry

    lax.fori_loop(0, NPAIR, pair_body, 0)
    out_dma(NCH - 2, out0, sem_o0).wait()
    out_dma(NCH - 1, out1, sem_o1).wait()


@jax.jit
def _intra_agg(features, ids_flat, self_feats):
    mesh = plsc.VectorSubcoreMesh(core_axis_name="c", subcore_axis_name="s")
    f = functools.partial(
        pl.kernel,
        mesh=mesh,
        compiler_params=pltpu.CompilerParams(needs_layout_passes=False),
        out_type=jax.ShapeDtypeStruct((B, 2 * D), jnp.float32),
        scratch_types=[
            pltpu.VMEM((RPW * K,), jnp.int32),
            pltpu.VMEM((N_NODES,), jnp.int32),
            pltpu.VMEM((RPW, L), jnp.float32),
            pltpu.VMEM((RPW, L), jnp.float32),
            pltpu.VMEM((R * K, D), jnp.float32),
            pltpu.VMEM((R * K, D), jnp.float32),
            pltpu.VMEM((R, D), jnp.float32),
            pltpu.VMEM((R, D), jnp.float32),
            pltpu.VMEM((R, 2 * D), jnp.float32),
            pltpu.VMEM((R, 2 * D), jnp.float32),
            pltpu.SemaphoreType.DMA,
            pltpu.SemaphoreType.DMA,
            pltpu.SemaphoreType.DMA,
            pltpu.SemaphoreType.DMA,
            pltpu.SemaphoreType.DMA,
            pltpu.SemaphoreType.DMA,
            pltpu.SemaphoreType.DMA,
            pltpu.SemaphoreType.DMA,
        ],
    )(_sc_body)
    return f(features, ids_flat, self_feats)


def kernel(features, nodes, to_neighs_list, self_feats):
    del nodes
    ids_flat = to_neighs_list.astype(jnp.int32).reshape(-1)
    return _intra_agg(features, ids_flat, self_feats)


# speedup vs baseline: 3.3893x; 1.0093x over previous
import functools

import jax
import jax.numpy as jnp
from jax import lax
from jax.experimental import pallas as pl
from jax.experimental.pallas import tpu as pltpu
from jax.experimental.pallas import tpu_sc as plsc

N_NODES = 10000
D = 512
B = 2048
K = 32
L = 16
NW = 32
RPW = B // NW
R = 2
NCH = RPW // R
NPAIR = NCH // 2
CPD = D // L


def _sc_body(feat_hbm, ids_hbm, self_hbm, out_hbm,
             ids_v, table_v, nd_v, inv_v,
             rows0, rows1, self0, self1, out0, out1,
             sem_r0, sem_r1, sem_s0, sem_s1, sem_o0, sem_o1):
    cid = lax.axis_index("c")
    sid = lax.axis_index("s")
    wid = sid * 2 + cid
    row0 = wid * RPW

    pltpu.sync_copy(ids_hbm.at[pl.ds(row0 * K, RPW * K)], ids_v)

    pos_a = lax.iota(jnp.int32, L)
    pos_b = pos_a + L
    zf = jnp.zeros((L,), jnp.float32)
    zi = jnp.zeros((L,), jnp.int32)

    def dd_body(r, carry):
        base = r * K
        a = ids_v[pl.ds(base, L)]
        b = ids_v[pl.ds(base + L, L)]
        plsc.store_scatter(table_v, [a], pos_a)
        plsc.store_scatter(table_v, [b], pos_b)
        ga = plsc.load_gather(table_v, [a])
        gb = plsc.load_gather(table_v, [b])
        fa = ga == pos_a
        fb = gb == pos_b
        id0 = plsc.load_gather(ids_v, [zi + base])
        ids_v[pl.ds(base, L)] = jnp.where(fa, a, id0)
        ids_v[pl.ds(base + L, L)] = jnp.where(fb, b, id0)
        fa_f = jnp.where(fa, 1.0, 0.0).astype(jnp.float32)
        fb_f = jnp.where(fb, 1.0, 0.0).astype(jnp.float32)
        n_unique = jnp.sum(fa_f) + jnp.sum(fb_f)
        nd_v[r, pl.ds(0, L)] = (K - n_unique) + zf
        inv_v[r, pl.ds(0, L)] = (1.0 + zf) / (n_unique + zf)
        return carry

    lax.fori_loop(0, RPW, dd_body, 0)

    def rows_dma(ch, buf, sem):
        idx = ids_v.at[pl.ds(ch * R * K, R * K)]
        return pltpu.make_async_copy(feat_hbm.at[idx], buf, sem)

    def self_dma(ch, buf, sem):
        return pltpu.make_async_copy(
            self_hbm.at[pl.ds(row0 + ch * R, R)], buf, sem)

    UNROLL = 1

    def compute(ch, rows_b, self_b, out_b):
        for rr in range(R):
            def acc_body(jj, acc):
                for u in range(UNROLL):
                    rbase = rr * K + jj * UNROLL + u
                    acc = tuple(acc[c] + rows_b[rbase, pl.ds(c * L, L)]
                                for c in range(CPD))
                return acc

            acc0 = tuple(jnp.zeros((L,), jnp.float32) for _ in range(CPD))
            acc = lax.fori_loop(0, K // UNROLL, acc_body, acc0)

            r = ch * R + rr
            nd = nd_v[r, pl.ds(0, L)]
            inv = inv_v[r, pl.ds(0, L)]
            for c in range(CPD):
                r0c = rows_b[rr * K, pl.ds(c * L, L)]
                aggc = (acc[c] - nd * r0c) * inv
                out_b[rr, pl.ds(c * L, L)] = self_b[rr, pl.ds(c * L, L)] - aggc
                out_b[rr, pl.ds(D + c * L, L)] = aggc

    def out_dma(ch, buf, sem):
        return pltpu.make_async_copy(
            buf, out_hbm.at[pl.ds(row0 + ch * R, R)], sem)

    rows_dma(0, rows0, sem_r0).start()
    self_dma(0, self0, sem_s0).start()

    def pair_body(i2, carry):
        ch0 = i2 * 2
        ch1 = ch0 + 1
        rows_dma(ch1, rows1, sem_r1).start()
        self_dma(ch1, self1, sem_s1).start()

        rows_dma(ch0, rows0, sem_r0).wait()
        self_dma(ch0, self0, sem_s0).wait()

        @pl.when(i2 > 0)
        def _():
            out_dma(ch0 - 2, out0, sem_o0).wait()

        compute(ch0, rows0, self0, out0)
        out_dma(ch0, out0, sem_o0).start()

        @pl.when(i2 < NPAIR - 1)
        def _():
            rows_dma(ch0 + 2, rows0, sem_r0).start()
            self_dma(ch0 + 2, self0, sem_s0).start()

        rows_dma(ch1, rows1, sem_r1).wait()
        self_dma(ch1, self1, sem_s1).wait()

        @pl.when(i2 > 0)
        def _():
            out_dma(ch1 - 2, out1, sem_o1).wait()

        compute(ch1, rows1, self1, out1)
        out_dma(ch1, out1, sem_o1).start()
        return carry

    lax.fori_loop(0, NPAIR, pair_body, 0)
    out_dma(NCH - 2, out0, sem_o0).wait()
    out_dma(NCH - 1, out1, sem_o1).wait()


@jax.jit
def _intra_agg(features, ids_flat, self_feats):
    mesh = plsc.VectorSubcoreMesh(core_axis_name="c", subcore_axis_name="s")
    f = functools.partial(
        pl.kernel,
        mesh=mesh,
        compiler_params=pltpu.CompilerParams(needs_layout_passes=False),
        out_type=jax.ShapeDtypeStruct((B, 2 * D), jnp.float32),
        scratch_types=[
            pltpu.VMEM((RPW * K,), jnp.int32),
            pltpu.VMEM((N_NODES,), jnp.int32),
            pltpu.VMEM((RPW, L), jnp.float32),
            pltpu.VMEM((RPW, L), jnp.float32),
            pltpu.VMEM((R * K, D), jnp.float32),
            pltpu.VMEM((R * K, D), jnp.float32),
            pltpu.VMEM((R, D), jnp.float32),
            pltpu.VMEM((R, D), jnp.float32),
            pltpu.VMEM((R, 2 * D), jnp.float32),
            pltpu.VMEM((R, 2 * D), jnp.float32),
            pltpu.SemaphoreType.DMA,
            pltpu.SemaphoreType.DMA,
            pltpu.SemaphoreType.DMA,
            pltpu.SemaphoreType.DMA,
            pltpu.SemaphoreType.DMA,
            pltpu.SemaphoreType.DMA,
        ],
    )(_sc_body)
    return f(features, ids_flat, self_feats)


def kernel(features, nodes, to_neighs_list, self_feats):
    del nodes
    ids_flat = to_neighs_list.astype(jnp.int32).reshape(-1)
    return _intra_agg(features, ids_flat, self_feats)
